# speedup vs baseline: 22.3290x; 22.3290x over previous
import functools

import jax
import jax.numpy as jnp
from jax import lax
from jax.experimental import pallas as pl
from jax.experimental.pallas import tpu as pltpu
from jax.experimental.pallas import tpu_sc as plsc

N = 10000
D = 128
E = 320000
NC, NS = 2, 16
NW = NC * NS
NP = 10240
PAD_SINK = NP - N
CB = 128
CH = 80
EP = NW * CH * CB
RPT = NP // NS
ZR = 64


def _mesh():
    return plsc.VectorSubcoreMesh(core_axis_name="c", subcore_axis_name="s")


def _make_deg():
    @functools.partial(
        pl.kernel,
        out_type=[jax.ShapeDtypeStruct((NP,), jnp.float32)] * 2,
        mesh=_mesh(),
        scratch_types=[
            pltpu.VMEM((CH, CB), jnp.int32),
            pltpu.VMEM((CB,), jnp.float32),
            pltpu.VMEM((RPT,), jnp.float32),
            pltpu.VMEM_SHARED((NP,), jnp.float32),
            pltpu.SemaphoreType.DMA,
        ],
    )
    def deg(dstw, out0, out1, dst_v, ones_v, zvec, accd, ssem):
        c = lax.axis_index("c")
        s = lax.axis_index("s")
        wid = c * NS + s

        def fill_ones(k, carry):
            ones_v[pl.ds(k * 16, 16)] = jnp.full((16,), 1.0, jnp.float32)
            return carry

        lax.fori_loop(0, CB // 16, fill_ones, 0)

        def fill_zero(k, carry):
            zvec[pl.ds(k * 16, 16)] = jnp.zeros((16,), jnp.float32)
            return carry

        lax.fori_loop(0, RPT // 16, fill_zero, 0)
        pltpu.sync_copy(zvec, accd.at[pl.ds(s * RPT, RPT)])
        pltpu.sync_copy(dstw.at[wid], dst_v)
        plsc.subcore_barrier()

        def chunk(j, carry):
            pltpu.async_copy(ones_v, accd.at[dst_v.at[j]], ssem, add=True).wait()
            return carry

        lax.fori_loop(0, CH, chunk, 0)
        plsc.subcore_barrier()

        @pl.when(s == 0)
        def _dump():
            @pl.when(c == 0)
            def _d0():
                pltpu.sync_copy(accd, out0)

            @pl.when(c == 1)
            def _d1():
                pltpu.sync_copy(accd, out1)

    return deg


def _make_agg():
    @functools.partial(
        pl.kernel,
        out_type=[jax.ShapeDtypeStruct((NP, D), jnp.float32)] * 2,
        mesh=_mesh(),
        scratch_types=[
            pltpu.VMEM((CH, CB), jnp.int32),
            pltpu.VMEM((CH, CB), jnp.int32),
            pltpu.VMEM((CB, D), jnp.float32),
            pltpu.VMEM((ZR, D), jnp.float32),
            pltpu.VMEM_SHARED((NP, D), jnp.float32),
            pltpu.SemaphoreType.DMA,
            pltpu.SemaphoreType.DMA,
        ],
    )
    def agg(g_hbm, srcw, dstw, out0, out1, src_v, dst_v, rows_v, zero_v,
            acc, gsem, ssem):
        c = lax.axis_index("c")
        s = lax.axis_index("s")
        wid = c * NS + s

        def fill_zero(k, carry):
            zero_v[k // 8, pl.ds((k % 8) * 16, 16)] = jnp.zeros((16,), jnp.float32)
            return carry

        lax.fori_loop(0, ZR * (D // 16), fill_zero, 0)

        def zero_acc(j, carry):
            pltpu.sync_copy(zero_v, acc.at[pl.ds(s * RPT + j * ZR, ZR)])
            return carry

        lax.fori_loop(0, RPT // ZR, zero_acc, 0)
        pltpu.sync_copy(srcw.at[wid], src_v)
        pltpu.sync_copy(dstw.at[wid], dst_v)
        plsc.subcore_barrier()

        def chunk(j, carry):
            pltpu.async_copy(g_hbm.at[src_v.at[j]], rows_v, gsem).wait()
            pltpu.async_copy(rows_v, acc.at[dst_v.at[j]], ssem, add=True).wait()
            return carry

        lax.fori_loop(0, CH, chunk, 0)
        plsc.subcore_barrier()

        @pl.when(c == 0)
        def _d0():
            pltpu.sync_copy(acc.at[pl.ds(s * RPT, RPT)],
                            out0.at[pl.ds(s * RPT, RPT)])

        @pl.when(c == 1)
        def _d1():
            pltpu.sync_copy(acc.at[pl.ds(s * RPT, RPT)],
                            out1.at[pl.ds(s * RPT, RPT)])

    return agg


_R = 1024


def _mm_body(x_ref, w_ref, o_ref):
    o_ref[...] = jnp.dot(x_ref[...], w_ref[...],
                         preferred_element_type=jnp.float32)


def _matmul(xp, w):
    return pl.pallas_call(
        _mm_body,
        grid=(NP // _R,),
        in_specs=[
            pl.BlockSpec((_R, D), lambda i: (i, 0)),
            pl.BlockSpec((D, D), lambda i: (0, 0)),
        ],
        out_specs=pl.BlockSpec((_R, D), lambda i: (i, 0)),
        out_shape=jax.ShapeDtypeStruct((NP, D), jnp.float32),
    )(xp, w)


def _dinv(d0, d1):
    return lax.rsqrt(jnp.maximum(d0 + d1 + 1.0, 1.0))


def _g1_body(h_ref, d0_ref, d1_ref, o_ref):
    o_ref[...] = h_ref[...] * _dinv(d0_ref[...], d1_ref[...])


def _scale(h, d0, d1):
    return pl.pallas_call(
        _g1_body,
        grid=(NP // _R,),
        in_specs=[
            pl.BlockSpec((_R, D), lambda i: (i, 0)),
            pl.BlockSpec((_R, 1), lambda i: (i, 0)),
            pl.BlockSpec((_R, 1), lambda i: (i, 0)),
        ],
        out_specs=pl.BlockSpec((_R, D), lambda i: (i, 0)),
        out_shape=jax.ShapeDtypeStruct((NP, D), jnp.float32),
    )(h, d0, d1)


def _mid_body(z0_ref, z1_ref, g1_ref, d0_ref, d1_ref, b1_ref, a1_ref, w2_ref,
              o_ref):
    i = pl.program_id(0)
    dinv = _dinv(d0_ref[...], d1_ref[...])
    z = z0_ref[...] + z1_ref[...] + g1_ref[...]
    out1 = z * dinv + b1_ref[...]
    h = jnp.where(out1 >= 0.0, out1, a1_ref[...] * out1)
    g2 = jnp.dot(h, w2_ref[...], preferred_element_type=jnp.float32) * dinv
    rows = i * _R + lax.broadcasted_iota(jnp.int32, (_R, 1), 0)
    o_ref[...] = jnp.where(rows < N, g2, 0.0)


def _mid(z0, z1, g1, d0, d1, b1, a1, w2):
    return pl.pallas_call(
        _mid_body,
        grid=(NP // _R,),
        in_specs=[
            pl.BlockSpec((_R, D), lambda i: (i, 0)),
            pl.BlockSpec((_R, D), lambda i: (i, 0)),
            pl.BlockSpec((_R, D), lambda i: (i, 0)),
            pl.BlockSpec((_R, 1), lambda i: (i, 0)),
            pl.BlockSpec((_R, 1), lambda i: (i, 0)),
            pl.BlockSpec((1, D), lambda i: (0, 0)),
            pl.BlockSpec((1, D), lambda i: (0, 0)),
            pl.BlockSpec((D, D), lambda i: (0, 0)),
        ],
        out_specs=pl.BlockSpec((_R, D), lambda i: (i, 0)),
        out_shape=jax.ShapeDtypeStruct((NP, D), jnp.float32),
    )(z0, z1, g1, d0, d1, b1, a1, w2)


def _final_body(z0_ref, z1_ref, g2_ref, d0_ref, d1_ref, b2_ref, o_ref):
    dinv = _dinv(d0_ref[...], d1_ref[...])
    z = z0_ref[...] + z1_ref[...] + g2_ref[...]
    o_ref[...] = z * dinv + b2_ref[...]


def _final(z0, z1, g2, d0, d1, b2):
    return pl.pallas_call(
        _final_body,
        grid=(NP // _R,),
        in_specs=[
            pl.BlockSpec((_R, D), lambda i: (i, 0)),
            pl.BlockSpec((_R, D), lambda i: (i, 0)),
            pl.BlockSpec((_R, D), lambda i: (i, 0)),
            pl.BlockSpec((_R, 1), lambda i: (i, 0)),
            pl.BlockSpec((_R, 1), lambda i: (i, 0)),
            pl.BlockSpec((1, D), lambda i: (0, 0)),
        ],
        out_specs=pl.BlockSpec((_R, D), lambda i: (i, 0)),
        out_shape=jax.ShapeDtypeStruct((NP, D), jnp.float32),
    )(z0, z1, g2, d0, d1, b2)


def kernel(x, edge_index, W1, b1, a1, W2, b2):
    ei = edge_index.astype(jnp.int32)
    pad_idx = N + (jnp.arange(EP - E, dtype=jnp.int32) % PAD_SINK)
    srcw = jnp.concatenate([ei[0], pad_idx]).reshape(NW, CH, CB)
    dstw = jnp.concatenate([ei[1], pad_idx]).reshape(NW, CH, CB)
    xp = jnp.pad(x, ((0, NP - N), (0, 0)))

    deg0, deg1 = _make_deg()(dstw)
    h1 = _matmul(xp, W1)
    d0 = deg0.reshape(NP, 1)
    d1 = deg1.reshape(NP, 1)
    g1 = _scale(h1, d0, d1)

    agg = _make_agg()
    z1a, z1b = agg(g1, srcw, dstw)
    g2 = _mid(z1a, z1b, g1, d0, d1, b1.reshape(1, D), a1.reshape(1, D), W2)
    z2a, z2b = agg(g2, srcw, dstw)
    out = _final(z2a, z2b, g2, d0, d1, b2.reshape(1, D))
    return out[:N]


# speedup vs baseline: 26.8727x; 1.2035x over previous
import functools

import jax
import jax.numpy as jnp
from jax import lax
from jax.experimental import pallas as pl
from jax.experimental.pallas import tpu as pltpu
from jax.experimental.pallas import tpu_sc as plsc

N = 10000
D = 128
E = 320000
NC, NS = 2, 16
NW = NC * NS
NP = 10240
PAD_SINK = NP - N
CB = 128
CH = 80
EP = NW * CH * CB
RPT = NP // NS
IR = 4


def _mesh():
    return plsc.VectorSubcoreMesh(core_axis_name="c", subcore_axis_name="s")


def _make_deg():
    @functools.partial(
        pl.kernel,
        out_type=jax.ShapeDtypeStruct((NC, NP), jnp.float32),
        mesh=_mesh(),
        scratch_types=[
            pltpu.VMEM((CH, 2, CB), jnp.int32),
            pltpu.VMEM((CB,), jnp.float32),
            pltpu.VMEM((RPT,), jnp.float32),
            pltpu.VMEM_SHARED((NP,), jnp.float32),
            pltpu.SemaphoreType.DMA,
        ],
    )
    def deg(sdw, out, idx_v, ones_v, zvec, accd, ssem):
        c = lax.axis_index("c")
        s = lax.axis_index("s")
        wid = c * NS + s

        def fill_ones(k, carry):
            ones_v[pl.ds(k * 16, 16)] = jnp.full((16,), 1.0, jnp.float32)
            return carry

        lax.fori_loop(0, CB // 16, fill_ones, 0)

        def fill_zero(k, carry):
            zvec[pl.ds(k * 16, 16)] = jnp.zeros((16,), jnp.float32)
            return carry

        lax.fori_loop(0, RPT // 16, fill_zero, 0)
        pltpu.sync_copy(zvec, accd.at[pl.ds(s * RPT, RPT)])
        pltpu.sync_copy(sdw.at[wid], idx_v)
        plsc.subcore_barrier()

        def chunk(j, carry):
            pltpu.async_copy(ones_v, accd.at[idx_v.at[j, 1]], ssem,
                             add=True).wait()
            return carry

        lax.fori_loop(0, CH, chunk, 0)
        plsc.subcore_barrier()

        pltpu.sync_copy(accd.at[pl.ds(s * RPT, RPT)],
                        out.at[c, pl.ds(s * RPT, RPT)])

    return deg


def _make_agg():
    @functools.partial(
        pl.kernel,
        out_type=jax.ShapeDtypeStruct((NC, NP, D), jnp.float32),
        mesh=_mesh(),
        scratch_types=[
            pltpu.VMEM((IR, 2, CB), jnp.int32),
            pltpu.VMEM((2, CB, D), jnp.float32),
            pltpu.VMEM_SHARED((NP, D), jnp.float32),
            pltpu.SemaphoreType.DMA,
            pltpu.SemaphoreType.DMA,
            pltpu.SemaphoreType.DMA,
            pltpu.SemaphoreType.DMA,
        ],
    )
    def agg(g_hbm, sdw, out, idx_v, rows_v, acc, isem, gsem,
            ssem0, ssem1):
        c = lax.axis_index("c")
        s = lax.axis_index("s")
        wid = c * NS + s

        def fill_zero(k, carry):
            rows_v[0, k // 8, pl.ds((k % 8) * 16, 16)] = jnp.zeros(
                (16,), jnp.float32)
            return carry

        lax.fori_loop(0, CB * (D // 16), fill_zero, 0)

        def zero_acc(j, carry):
            pltpu.sync_copy(rows_v.at[0], acc.at[pl.ds(s * RPT + j * CB, CB)])
            return carry

        lax.fori_loop(0, RPT // CB, zero_acc, 0)
        plsc.subcore_barrier()

        ssems = (ssem0, ssem1)

        def idx_dma(j):
            return pltpu.make_async_copy(sdw.at[wid, j],
                                         idx_v.at[lax.rem(j, IR)], isem)

        def gather(j, b):
            return pltpu.make_async_copy(
                g_hbm.at[idx_v.at[lax.rem(j, IR), 0]], rows_v.at[b], gsem)

        def scatter(j, b):
            return pltpu.make_async_copy(
                rows_v.at[b], acc.at[idx_v.at[lax.rem(j, IR), 1]], ssems[b])

        idx_dma(0).start()
        idx_dma(1).start()
        idx_dma(2).start()
        idx_dma(0).wait()
        pltpu.async_copy(g_hbm.at[idx_v.at[0, 0]], rows_v.at[0], gsem)

        def pipe(i, carry):
            for b in range(2):
                j = 2 * i + b
                gather(j, b).wait()
                pltpu.async_copy(rows_v.at[b],
                                 acc.at[idx_v.at[lax.rem(j, IR), 1]],
                                 ssems[b], add=True)

                @pl.when(j >= 1)
                def _drain_other():
                    scatter(j - 1, 1 - b).wait()

                @pl.when(j + 1 < CH)
                def _next_gather():
                    idx_dma(j + 1).wait()
                    gather(j + 1, 1 - b).start()

                @pl.when(j + 3 < CH)
                def _next_idx():
                    idx_dma(j + 3).start()
            return carry

        lax.fori_loop(0, CH // 2, pipe, 0)
        scatter(CH - 1, 1).wait()
        plsc.subcore_barrier()
        pltpu.sync_copy(acc.at[pl.ds(s * RPT, RPT)],
                        out.at[c, pl.ds(s * RPT, RPT)])

    return agg


_R = 1024


def _mm_body(x_ref, w_ref, o_ref):
    o_ref[...] = jnp.dot(x_ref[...], w_ref[...],
                         preferred_element_type=jnp.float32)


def _matmul(xp, w):
    return pl.pallas_call(
        _mm_body,
        grid=(NP // _R,),
        in_specs=[
            pl.BlockSpec((_R, D), lambda i: (i, 0)),
            pl.BlockSpec((D, D), lambda i: (0, 0)),
        ],
        out_specs=pl.BlockSpec((_R, D), lambda i: (i, 0)),
        out_shape=jax.ShapeDtypeStruct((NP, D), jnp.float32),
    )(xp, w)


def _dinv(d0, d1):
    return lax.rsqrt(jnp.maximum(d0 + d1 + 1.0, 1.0))


def _g1_body(h_ref, d0_ref, d1_ref, o_ref):
    o_ref[...] = h_ref[...] * _dinv(d0_ref[...], d1_ref[...])


def _scale(h, d0, d1):
    return pl.pallas_call(
        _g1_body,
        grid=(NP // _R,),
        in_specs=[
            pl.BlockSpec((_R, D), lambda i: (i, 0)),
            pl.BlockSpec((_R, 1), lambda i: (i, 0)),
            pl.BlockSpec((_R, 1), lambda i: (i, 0)),
        ],
        out_specs=pl.BlockSpec((_R, D), lambda i: (i, 0)),
        out_shape=jax.ShapeDtypeStruct((NP, D), jnp.float32),
    )(h, d0, d1)


def _mid_body(z0_ref, z1_ref, g1_ref, d0_ref, d1_ref, b1_ref, a1_ref, w2_ref,
              o_ref):
    i = pl.program_id(0)
    dinv = _dinv(d0_ref[...], d1_ref[...])
    z = z0_ref[...] + z1_ref[...] + g1_ref[...]
    out1 = z * dinv + b1_ref[...]
    h = jnp.where(out1 >= 0.0, out1, a1_ref[...] * out1)
    g2 = jnp.dot(h, w2_ref[...], preferred_element_type=jnp.float32) * dinv
    rows = i * _R + lax.broadcasted_iota(jnp.int32, (_R, 1), 0)
    o_ref[...] = jnp.where(rows < N, g2, 0.0)


def _mid(z0, z1, g1, d0, d1, b1, a1, w2):
    return pl.pallas_call(
        _mid_body,
        grid=(NP // _R,),
        in_specs=[
            pl.BlockSpec((_R, D), lambda i: (i, 0)),
            pl.BlockSpec((_R, D), lambda i: (i, 0)),
            pl.BlockSpec((_R, D), lambda i: (i, 0)),
            pl.BlockSpec((_R, 1), lambda i: (i, 0)),
            pl.BlockSpec((_R, 1), lambda i: (i, 0)),
            pl.BlockSpec((1, D), lambda i: (0, 0)),
            pl.BlockSpec((1, D), lambda i: (0, 0)),
            pl.BlockSpec((D, D), lambda i: (0, 0)),
        ],
        out_specs=pl.BlockSpec((_R, D), lambda i: (i, 0)),
        out_shape=jax.ShapeDtypeStruct((NP, D), jnp.float32),
    )(z0, z1, g1, d0, d1, b1, a1, w2)


def _final_body(z0_ref, z1_ref, g2_ref, d0_ref, d1_ref, b2_ref, o_ref):
    dinv = _dinv(d0_ref[...], d1_ref[...])
    z = z0_ref[...] + z1_ref[...] + g2_ref[...]
    o_ref[...] = z * dinv + b2_ref[...]


def _final(z0, z1, g2, d0, d1, b2):
    return pl.pallas_call(
        _final_body,
        grid=(NP // _R,),
        in_specs=[
            pl.BlockSpec((_R, D), lambda i: (i, 0)),
            pl.BlockSpec((_R, D), lambda i: (i, 0)),
            pl.BlockSpec((_R, D), lambda i: (i, 0)),
            pl.BlockSpec((_R, 1), lambda i: (i, 0)),
            pl.BlockSpec((_R, 1), lambda i: (i, 0)),
            pl.BlockSpec((1, D), lambda i: (0, 0)),
        ],
        out_specs=pl.BlockSpec((_R, D), lambda i: (i, 0)),
        out_shape=jax.ShapeDtypeStruct((NP, D), jnp.float32),
    )(z0, z1, g2, d0, d1, b2)


def kernel(x, edge_index, W1, b1, a1, W2, b2):
    ei = edge_index.astype(jnp.int32)
    pad_idx = N + (jnp.arange(EP - E, dtype=jnp.int32) % PAD_SINK)
    srcw = jnp.concatenate([ei[0], pad_idx]).reshape(NW, CH, CB)
    dstw = jnp.concatenate([ei[1], pad_idx]).reshape(NW, CH, CB)
    sdw = jnp.stack([srcw, dstw], axis=2)
    xp = jnp.pad(x, ((0, NP - N), (0, 0)))

    degs = _make_deg()(sdw)
    deg0, deg1 = degs[0], degs[1]
    h1 = _matmul(xp, W1)
    d0 = deg0.reshape(NP, 1)
    d1 = deg1.reshape(NP, 1)
    g1 = _scale(h1, d0, d1)

    agg = _make_agg()
    z1 = agg(g1, sdw)
    z1a, z1b = z1[0], z1[1]
    g2 = _mid(z1a, z1b, g1, d0, d1, b1.reshape(1, D), a1.reshape(1, D), W2)
    z2 = agg(g2, sdw)
    z2a, z2b = z2[0], z2[1]
    out = _final(z2a, z2b, g2, d0, d1, b2.reshape(1, D))
    return out[:N]


# speedup vs baseline: 27.3066x; 1.0161x over previous
import functools

import jax
import jax.numpy as jnp
from jax import lax
from jax.experimental import pallas as pl
from jax.experimental.pallas import tpu as pltpu
from jax.experimental.pallas import tpu_sc as plsc

N = 10000
D = 128
E = 320000
NC, NS = 2, 16
NW = NC * NS
NP = 10240
PAD_SINK = NP - N
CB = 128
CH = 80
EP = NW * CH * CB
RPT = NP // NS
IR = 4


def _mesh():
    return plsc.VectorSubcoreMesh(core_axis_name="c", subcore_axis_name="s")


def _make_deg():
    @functools.partial(
        pl.kernel,
        out_type=jax.ShapeDtypeStruct((NC, NP), jnp.float32),
        mesh=_mesh(),
        scratch_types=[
            pltpu.VMEM((CH, 2, CB), jnp.int32),
            pltpu.VMEM((CB,), jnp.float32),
            pltpu.VMEM((RPT,), jnp.float32),
            pltpu.VMEM_SHARED((NP,), jnp.float32),
            pltpu.SemaphoreType.DMA,
        ],
    )
    def deg(sdw, out, idx_v, ones_v, zvec, accd, ssem):
        c = lax.axis_index("c")
        s = lax.axis_index("s")
        wid = c * NS + s

        def fill_ones(k, carry):
            ones_v[pl.ds(k * 16, 16)] = jnp.full((16,), 1.0, jnp.float32)
            return carry

        lax.fori_loop(0, CB // 16, fill_ones, 0)

        def fill_zero(k, carry):
            zvec[pl.ds(k * 16, 16)] = jnp.zeros((16,), jnp.float32)
            return carry

        lax.fori_loop(0, RPT // 16, fill_zero, 0)
        pltpu.sync_copy(zvec, accd.at[pl.ds(s * RPT, RPT)])
        pltpu.sync_copy(sdw.at[wid], idx_v)
        plsc.subcore_barrier()

        def chunk(j, carry):
            pltpu.async_copy(ones_v, accd.at[idx_v.at[j, 1]], ssem,
                             add=True).wait()
            return carry

        lax.fori_loop(0, CH, chunk, 0)
        plsc.subcore_barrier()

        pltpu.sync_copy(accd.at[pl.ds(s * RPT, RPT)],
                        out.at[c, pl.ds(s * RPT, RPT)])

    return deg


def _make_agg():
    @functools.partial(
        pl.kernel,
        out_type=jax.ShapeDtypeStruct((NC, NP, D), jnp.float32),
        mesh=_mesh(),
        scratch_types=[
            pltpu.VMEM((IR, 2, CB), jnp.int32),
            pltpu.VMEM((2, CB, D), jnp.float32),
            pltpu.VMEM_SHARED((NP, D), jnp.float32),
            pltpu.SemaphoreType.DMA,
            pltpu.SemaphoreType.DMA,
            pltpu.SemaphoreType.DMA,
            pltpu.SemaphoreType.DMA,
        ],
    )
    def agg(g_hbm, sdw, out, idx_v, rows_v, acc, isem, gsem,
            ssem0, ssem1):
        c = lax.axis_index("c")
        s = lax.axis_index("s")
        wid = c * NS + s

        def fill_zero(k, carry):
            rows_v[0, k // 8, pl.ds((k % 8) * 16, 16)] = jnp.zeros(
                (16,), jnp.float32)
            return carry

        lax.fori_loop(0, CB * (D // 16), fill_zero, 0)

        def zero_acc(j, carry):
            pltpu.sync_copy(rows_v.at[0], acc.at[pl.ds(s * RPT + j * CB, CB)])
            return carry

        lax.fori_loop(0, RPT // CB, zero_acc, 0)
        plsc.subcore_barrier()

        ssems = (ssem0, ssem1)

        def idx_dma(j):
            return pltpu.make_async_copy(sdw.at[wid, j],
                                         idx_v.at[lax.rem(j, IR)], isem)

        def gather(j, b):
            return pltpu.make_async_copy(
                g_hbm.at[idx_v.at[lax.rem(j, IR), 0]], rows_v.at[b], gsem)

        def scatter(j, b):
            return pltpu.make_async_copy(
                rows_v.at[b], acc.at[idx_v.at[lax.rem(j, IR), 1]], ssems[b])

        idx_dma(0).start()
        idx_dma(1).start()
        idx_dma(2).start()
        idx_dma(0).wait()
        pltpu.async_copy(g_hbm.at[idx_v.at[0, 0]], rows_v.at[0], gsem)

        def pipe(i, carry):
            for b in range(2):
                j = 2 * i + b
                gather(j, b).wait()
                pltpu.async_copy(rows_v.at[b],
                                 acc.at[idx_v.at[lax.rem(j, IR), 1]],
                                 ssems[b], add=True)

                @pl.when(j >= 1)
                def _drain_other():
                    scatter(j - 1, 1 - b).wait()

                @pl.when(j + 1 < CH)
                def _next_gather():
                    idx_dma(j + 1).wait()
                    gather(j + 1, 1 - b).start()

                @pl.when(j + 3 < CH)
                def _next_idx():
                    idx_dma(j + 3).start()
            return carry

        lax.fori_loop(0, CH // 2, pipe, 0)
        scatter(CH - 1, 1).wait()
        plsc.subcore_barrier()
        pltpu.sync_copy(acc.at[pl.ds(s * RPT, RPT)],
                        out.at[c, pl.ds(s * RPT, RPT)])

    return agg


_R = 1024


def _dinv(d0, d1):
    return lax.rsqrt(jnp.maximum(d0 + d1 + 1.0, 1.0))


def _g1_body(x_ref, w_ref, d0_ref, d1_ref, o_ref):
    h = jnp.dot(x_ref[...], w_ref[...], preferred_element_type=jnp.float32)
    o_ref[...] = h * _dinv(d0_ref[...], d1_ref[...])


def _mm_scale(xp, w, d0, d1):
    return pl.pallas_call(
        _g1_body,
        grid=(NP // _R,),
        in_specs=[
            pl.BlockSpec((_R, D), lambda i: (i, 0)),
            pl.BlockSpec((D, D), lambda i: (0, 0)),
            pl.BlockSpec((_R, 1), lambda i: (i, 0)),
            pl.BlockSpec((_R, 1), lambda i: (i, 0)),
        ],
        out_specs=pl.BlockSpec((_R, D), lambda i: (i, 0)),
        out_shape=jax.ShapeDtypeStruct((NP, D), jnp.float32),
    )(xp, w, d0, d1)


def _mid_body(z0_ref, z1_ref, g1_ref, d0_ref, d1_ref, b1_ref, a1_ref, w2_ref,
              o_ref):
    i = pl.program_id(0)
    dinv = _dinv(d0_ref[...], d1_ref[...])
    z = z0_ref[...] + z1_ref[...] + g1_ref[...]
    out1 = z * dinv + b1_ref[...]
    h = jnp.where(out1 >= 0.0, out1, a1_ref[...] * out1)
    g2 = jnp.dot(h, w2_ref[...], preferred_element_type=jnp.float32) * dinv
    rows = i * _R + lax.broadcasted_iota(jnp.int32, (_R, 1), 0)
    o_ref[...] = jnp.where(rows < N, g2, 0.0)


def _mid(zf, g1, df, b1, a1, w2):
    nb = NP // _R
    return pl.pallas_call(
        _mid_body,
        grid=(nb,),
        in_specs=[
            pl.BlockSpec((_R, D), lambda i: (i, 0)),
            pl.BlockSpec((_R, D), lambda i: (i + nb, 0)),
            pl.BlockSpec((_R, D), lambda i: (i, 0)),
            pl.BlockSpec((_R, 1), lambda i: (i, 0)),
            pl.BlockSpec((_R, 1), lambda i: (i + nb, 0)),
            pl.BlockSpec((1, D), lambda i: (0, 0)),
            pl.BlockSpec((1, D), lambda i: (0, 0)),
            pl.BlockSpec((D, D), lambda i: (0, 0)),
        ],
        out_specs=pl.BlockSpec((_R, D), lambda i: (i, 0)),
        out_shape=jax.ShapeDtypeStruct((NP, D), jnp.float32),
    )(zf, zf, g1, df, df, b1, a1, w2)


def _final_body(z0_ref, z1_ref, g2_ref, d0_ref, d1_ref, b2_ref, o_ref):
    dinv = _dinv(d0_ref[...], d1_ref[...])
    z = z0_ref[...] + z1_ref[...] + g2_ref[...]
    o_ref[...] = z * dinv + b2_ref[...]


def _final(zf, g2, df, b2):
    nb = NP // _R
    return pl.pallas_call(
        _final_body,
        grid=(nb,),
        in_specs=[
            pl.BlockSpec((_R, D), lambda i: (i, 0)),
            pl.BlockSpec((_R, D), lambda i: (i + nb, 0)),
            pl.BlockSpec((_R, D), lambda i: (i, 0)),
            pl.BlockSpec((_R, 1), lambda i: (i, 0)),
            pl.BlockSpec((_R, 1), lambda i: (i + nb, 0)),
            pl.BlockSpec((1, D), lambda i: (0, 0)),
        ],
        out_specs=pl.BlockSpec((_R, D), lambda i: (i, 0)),
        out_shape=jax.ShapeDtypeStruct((NP, D), jnp.float32),
    )(zf, zf, g2, df, df, b2)


def kernel(x, edge_index, W1, b1, a1, W2, b2):
    ei = edge_index.astype(jnp.int32)
    pad_idx = N + (jnp.arange(EP - E, dtype=jnp.int32) % PAD_SINK)
    srcw = jnp.concatenate([ei[0], pad_idx]).reshape(NW, CH, CB)
    dstw = jnp.concatenate([ei[1], pad_idx]).reshape(NW, CH, CB)
    sdw = jnp.stack([srcw, dstw], axis=2)
    xp = jnp.pad(x, ((0, NP - N), (0, 0)))

    degs = _make_deg()(sdw)
    df = degs.reshape(NC * NP, 1)
    g1 = _mm_scale(xp, W1, df[:NP], df[NP:])

    agg = _make_agg()
    z1 = agg(g1, sdw).reshape(NC * NP, D)
    g2 = _mid(z1, g1, df, b1.reshape(1, D), a1.reshape(1, D), W2)
    z2 = agg(g2, sdw).reshape(NC * NP, D)
    out = _final(z2, g2, df, b2.reshape(1, D))
    return out[:N]
